# Initial kernel scaffold; baseline (speedup 1.0000x reference)
import jax
import jax.numpy as jnp
from jax.experimental import pallas as pl


def kernel(x, emb, fc1_w, fc1_b, fc2_w, fc2_b):
    raise NotImplementedError("write your pallas kernel here")


# speedup vs baseline: 1.6162x; 1.6162x over previous
import functools

import jax
import jax.numpy as jnp
from jax import lax
from jax.experimental import pallas as pl
from jax.experimental.pallas import tpu as pltpu
from jax.experimental.pallas import tpu_sc as plsc

VOCAB = 1000
EMB_DIM = 128
HID = 8
HID_P = 16
ROW = 128
VPAD = 1024
NC = 2
NS = 16
NW = NC * NS
CHUNK = 128


def _table_body(embp_ref, wa_ref, wb_ref, t_ref):
    dn = (((1,), (1,)), ((), ()))
    t_ref[0:VPAD, :] = lax.dot_general(
        embp_ref[...], wa_ref[...], dn, preferred_element_type=jnp.float32)
    t_ref[VPAD:2 * VPAD, :] = lax.dot_general(
        embp_ref[...], wb_ref[...], dn, preferred_element_type=jnp.float32)


def _build_table(embp, wa, wb):
    return pl.pallas_call(
        _table_body,
        out_shape=jax.ShapeDtypeStruct((2 * VPAD, ROW), jnp.float32),
    )(embp, wa, wb)


def _sc_gather(table, idx0, idx1, batch):
    bpw = batch // NW
    nch = bpw // CHUNK
    mesh = plsc.VectorSubcoreMesh(core_axis_name="c", subcore_axis_name="s")

    @functools.partial(
        pl.kernel, mesh=mesh,
        out_type=jax.ShapeDtypeStruct((batch, HID_P), jnp.float32),
        scratch_types=[
            pltpu.VMEM((nch, CHUNK), jnp.int32),
            pltpu.VMEM((nch, CHUNK), jnp.int32),
            pltpu.VMEM((CHUNK, ROW), jnp.float32),
            pltpu.VMEM((CHUNK, ROW), jnp.float32),
            pltpu.VMEM((bpw, HID_P), jnp.float32),
            pltpu.SemaphoreType.DMA,
        ],
    )
    def gather_k(table_hbm, idx0_hbm, idx1_hbm, out_hbm,
                 i0_v, i1_v, g0_v, g1_v, h_v, sem):
        wid = lax.axis_index("s") * NC + lax.axis_index("c")
        pltpu.sync_copy(idx0_hbm.at[pl.ds(wid * nch, nch)], i0_v)
        pltpu.sync_copy(idx1_hbm.at[pl.ds(wid * nch, nch)], i1_v)
        for j in range(nch):
            c0 = pltpu.async_copy(table_hbm.at[i0_v.at[j]], g0_v, sem)
            c1 = pltpu.async_copy(table_hbm.at[i1_v.at[j]], g1_v, sem)
            c0.wait()
            c1.wait()

            def body(i, carry, j=j):
                h_v[j * CHUNK + i, :] = g0_v[i, 0:HID_P] + g1_v[i, 0:HID_P]
                return carry

            lax.fori_loop(0, CHUNK, body, 0)
        pltpu.sync_copy(h_v, out_hbm.at[pl.ds(wid * bpw, bpw)])

    return gather_k(table, idx0, idx1)


def _mlp_body(h_ref, w2_ref, b1_ref, b2_ref, out_ref):
    ht = jnp.tanh(h_ref[...] + b1_ref[...])
    dn = (((1,), (1,)), ((), ()))
    acc = lax.dot_general(ht, w2_ref[...], dn, preferred_element_type=jnp.float32)
    out_ref[...] = acc + b2_ref[...]


def kernel(x, emb, fc1_w, fc1_b, fc2_w, fc2_b):
    x = x.astype(jnp.int32)
    batch = x.shape[0]
    tile = 1024

    embp = jnp.pad(emb, ((0, VPAD - VOCAB), (0, 0)))
    w1p = jnp.pad(fc1_w, ((0, ROW - HID), (0, 0)))
    table = _build_table(embp, w1p[:, :EMB_DIM], w1p[:, EMB_DIM:])

    idx0 = x[:, 0].reshape(batch // CHUNK, CHUNK)
    idx1 = (x[:, 1] + VPAD).reshape(batch // CHUNK, CHUNK)
    h = _sc_gather(table, idx0, idx1, batch)

    w2p = jnp.pad(fc2_w, ((0, 0), (0, HID_P - HID)))
    b1p = jnp.pad(fc1_b, (0, HID_P - HID)).reshape(1, HID_P)
    b2 = fc2_b.reshape(1, VOCAB)
    return pl.pallas_call(
        _mlp_body,
        grid=(batch // tile,),
        in_specs=[
            pl.BlockSpec((tile, HID_P), lambda i: (i, 0)),
            pl.BlockSpec((VOCAB, HID_P), lambda i: (0, 0)),
            pl.BlockSpec((1, HID_P), lambda i: (0, 0)),
            pl.BlockSpec((1, VOCAB), lambda i: (0, 0)),
        ],
        out_specs=pl.BlockSpec((tile, VOCAB), lambda i: (i, 0)),
        out_shape=jax.ShapeDtypeStruct((batch, VOCAB), jnp.float32),
    )(h, w2p, b1p, b2)
